# baseline (speedup 1.0000x reference)
import functools

import jax
import jax.numpy as jnp
from jax.experimental import pallas as pl
from jax.experimental.pallas import tpu as pltpu

_F32 = jnp.float32


def _rup(v, m):
    return -(-v // m) * m


def _bdiv(total, cap, step=128):
    if total < step:
        return total
    best = None
    d = step
    while d <= total:
        if total % d == 0 and d <= cap:
            best = d
        d += step
    assert best is not None, (total, cap, step)
    return best


def _mm(a, b, bias=None):
    M, K = a.shape
    K2, H = b.shape
    assert K == K2
    bm = _bdiv(M, 1024, step=8)
    have_bias = bias is not None

    def body(*refs):
        if have_bias:
            a_ref, b_ref, bias_ref, o_ref = refs
        else:
            a_ref, b_ref, o_ref = refs
        acc = jnp.dot(a_ref[...], b_ref[...], preferred_element_type=_F32,
                      precision=jax.lax.Precision.HIGHEST)
        if have_bias:
            acc = acc + bias_ref[...]
        o_ref[...] = acc

    in_specs = [
        pl.BlockSpec((bm, K), lambda i: (i, 0)),
        pl.BlockSpec((K, H), lambda i: (0, 0)),
    ]
    args = [a, b]
    if have_bias:
        in_specs.append(pl.BlockSpec((1, H), lambda i: (0, 0)))
        args.append(bias)
    return pl.pallas_call(
        body,
        grid=(M // bm,),
        in_specs=in_specs,
        out_specs=pl.BlockSpec((bm, H), lambda i: (i, 0)),
        out_shape=jax.ShapeDtypeStruct((M, H), _F32),
    )(*args)


def _transpose(a):
    n, m = a.shape
    bi = _bdiv(n, 512)
    bj = _bdiv(m, 512)

    def body(a_ref, o_ref):
        o_ref[...] = a_ref[...].T

    return pl.pallas_call(
        body,
        grid=(n // bi, m // bj),
        in_specs=[pl.BlockSpec((bi, bj), lambda i, j: (i, j))],
        out_specs=pl.BlockSpec((bj, bi), lambda i, j: (j, i)),
        out_shape=jax.ShapeDtypeStruct((m, n), a.dtype),
    )(a)


def _aug_mm(a, b):
    M, K = a.shape
    N2, K2 = b.shape
    assert K == K2
    bm = _bdiv(M, 512)
    bn = _bdiv(N2, 512)
    bk = _bdiv(K, 2048)
    nk = K // bk
    prec = (jax.lax.Precision.HIGHEST if a.dtype == _F32
            else jax.lax.Precision.DEFAULT)

    def body(a_ref, b_ref, o_ref, acc):
        i = pl.program_id(0)
        j = pl.program_id(1)
        k = pl.program_id(2)

        @pl.when(k == 0)
        def _():
            acc[...] = jnp.zeros_like(acc)

        acc[...] += jax.lax.dot_general(
            a_ref[...], b_ref[...], (((1,), (1,)), ((), ())),
            preferred_element_type=_F32, precision=prec)

        @pl.when(k == nk - 1)
        def _():
            r = jax.lax.broadcasted_iota(jnp.int32, (bm, bn), 0) + i * bm
            c = jax.lax.broadcasted_iota(jnp.int32, (bm, bn), 1) + j * bn
            o_ref[...] = jnp.where(r == c, 0.0, acc[...])

    return pl.pallas_call(
        body,
        grid=(M // bm, N2 // bn, nk),
        in_specs=[
            pl.BlockSpec((bm, bk), lambda i, j, k: (i, k)),
            pl.BlockSpec((bn, bk), lambda i, j, k: (j, k)),
        ],
        out_specs=pl.BlockSpec((bm, bn), lambda i, j, k: (i, j)),
        out_shape=jax.ShapeDtypeStruct((M, N2), _F32),
        scratch_shapes=[pltpu.VMEM((bm, bn), _F32)],
    )(a, b)


def _colsum_diag(a, with_diag):
    n = a.shape[0]
    bn = _bdiv(n, 512)
    bk = _bdiv(n, 512)
    nk = n // bk
    n_out = 2 if with_diag else 1

    def body(a_ref, *refs):
        outs = refs[:n_out]
        accs = refs[n_out:]
        k = pl.program_id(1)

        @pl.when(k == 0)
        def _():
            for acc in accs:
                acc[...] = jnp.zeros_like(acc)

        blk = a_ref[...]
        ones = jnp.ones((bk, 1), _F32)
        accs[0][...] += jax.lax.dot_general(
            blk, ones, (((0,), (0,)), ((), ())), preferred_element_type=_F32)
        if with_diag:
            j = pl.program_id(0)
            r = jax.lax.broadcasted_iota(jnp.int32, (bk, bn), 0) + k * bk
            c = jax.lax.broadcasted_iota(jnp.int32, (bk, bn), 1) + j * bn
            accs[1][...] += jax.lax.dot_general(
                jnp.where(r == c, blk, 0.0), ones, (((0,), (0,)), ((), ())),
                preferred_element_type=_F32)

        @pl.when(k == nk - 1)
        def _():
            for o, acc in zip(outs, accs):
                o[...] = acc[...]

    out_shape = [jax.ShapeDtypeStruct((n, 1), _F32)] * n_out
    res = pl.pallas_call(
        body,
        grid=(n // bn, nk),
        in_specs=[pl.BlockSpec((bk, bn), lambda j, k: (k, j))],
        out_specs=[pl.BlockSpec((bn, 1), lambda j, k: (j, 0))] * n_out,
        out_shape=out_shape,
        scratch_shapes=[pltpu.VMEM((bn, 1), _F32)] * n_out,
    )(a)
    if with_diag:
        return res[0], res[1]
    return res[0], None


def _conv(a, xw, cs, diag, bias, relu, real, lvl0):
    n = a.shape[0]
    H = xw.shape[1]
    bm = _bdiv(n, 512)
    bk = _bdiv(n, 512)
    nk = n // bk

    def dinv_t(cs_blk, d_blk):
        if lvl0:
            c = d_blk
            a2d = jnp.where(c > 0, c, 2.0)
            deg = cs_blk - c + a2d
            t = a2d - c
        else:
            deg = cs_blk + 2.0
            t = jnp.full_like(cs_blk, 2.0)
        dinv = jnp.where(deg > 0, 1.0 / jnp.sqrt(deg), 0.0)
        return dinv, t

    def body(*refs):
        if lvl0:
            (a_ref, xwk_ref, csk_ref, dk_ref, xwm_ref, csm_ref, dm_ref,
             b_ref, o_ref, acc) = refs
        else:
            (a_ref, xwk_ref, csk_ref, xwm_ref, csm_ref, b_ref, o_ref,
             acc) = refs
            dk_ref = dm_ref = None
        m = pl.program_id(0)
        k = pl.program_id(1)

        @pl.when(k == 0)
        def _():
            acc[...] = jnp.zeros_like(acc)

        dinv_k, _ = dinv_t(csk_ref[...], dk_ref[...] if lvl0 else None)
        v = xwk_ref[...] * dinv_k
        acc[...] += jax.lax.dot_general(
            a_ref[...], v, (((0,), (0,)), ((), ())),
            preferred_element_type=_F32,
            precision=jax.lax.Precision.HIGHEST)

        @pl.when(k == nk - 1)
        def _():
            dinv_m, t_m = dinv_t(csm_ref[...], dm_ref[...] if lvl0 else None)
            o = (acc[...] * dinv_m
                 + t_m * dinv_m * dinv_m * xwm_ref[...] + b_ref[...])
            if relu:
                o = jnp.maximum(o, 0.0)
            rows = jax.lax.broadcasted_iota(jnp.int32, (bm, 1), 0) + m * bm
            o_ref[...] = jnp.where(rows < real, o, 0.0)

    in_specs = [pl.BlockSpec((bk, bm), lambda m, k: (k, m)),
                pl.BlockSpec((bk, H), lambda m, k: (k, 0)),
                pl.BlockSpec((bk, 1), lambda m, k: (k, 0))]
    args = [a, xw, cs]
    if lvl0:
        in_specs.append(pl.BlockSpec((bk, 1), lambda m, k: (k, 0)))
        args.append(diag)
    in_specs += [pl.BlockSpec((bm, H), lambda m, k: (m, 0)),
                 pl.BlockSpec((bm, 1), lambda m, k: (m, 0))]
    args += [xw, cs]
    if lvl0:
        in_specs.append(pl.BlockSpec((bm, 1), lambda m, k: (m, 0)))
        args.append(diag)
    in_specs.append(pl.BlockSpec((1, H), lambda m, k: (0, 0)))
    args.append(bias)

    return pl.pallas_call(
        body,
        grid=(n // bm, nk),
        in_specs=in_specs,
        out_specs=pl.BlockSpec((bm, H), lambda m, k: (m, 0)),
        out_shape=jax.ShapeDtypeStruct((n, H), _F32),
        scratch_shapes=[pltpu.VMEM((bm, H), _F32)],
    )(*args)


def _rowdot(h, w, mode, real):
    n, H = h.shape
    bm = _bdiv(n, 1024, step=8)

    def body(h_ref, w_ref, o_ref):
        wv = w_ref[...]
        d = jnp.dot(h_ref[...], wv, preferred_element_type=_F32)
        if mode == "score":
            d = jnp.tanh(d * jax.lax.rsqrt(jnp.sum(wv * wv)))
            pad_val = -2.0
        else:
            pad_val = 0.0
        m = pl.program_id(0)
        rows = jax.lax.broadcasted_iota(jnp.int32, (bm, 1), 0) + m * bm
        o_ref[...] = jnp.where(rows < real, d, pad_val)

    return pl.pallas_call(
        body,
        grid=(n // bm,),
        in_specs=[pl.BlockSpec((bm, H), lambda m: (m, 0)),
                  pl.BlockSpec((H, 1), lambda m: (0, 0))],
        out_specs=pl.BlockSpec((bm, 1), lambda m: (m, 0)),
        out_shape=jax.ShapeDtypeStruct((n, 1), _F32),
    )(h, w)


_G = 8


def _gather_diag1(src, perm, real, n_out, dt):
    ns = src.shape[1]
    src3 = src.reshape(src.shape[0], 1, ns)

    def body(perm_ref, *refs):
        in_refs = refs[:_G]
        o_ref = refs[_G]
        i = pl.program_id(0)
        cid = jax.lax.broadcasted_iota(jnp.int32, (1, ns), 1)
        for g in range(_G):
            j = i * _G + g
            col = perm_ref[j]
            row = jnp.where(cid == col, 1.0, in_refs[g][0])
            row = jnp.where(j < real, row, 0.0)
            o_ref[g:g + 1, :] = row.astype(dt)

    def mk_map(g):
        return lambda i, pref: (pref[i * _G + g], 0, 0)

    grid_spec = pltpu.PrefetchScalarGridSpec(
        num_scalar_prefetch=1,
        grid=(n_out // _G,),
        in_specs=[pl.BlockSpec((1, 1, ns), mk_map(g)) for g in range(_G)],
        out_specs=pl.BlockSpec((_G, ns), lambda i, pref: (i, 0)),
    )
    return pl.pallas_call(
        body,
        grid_spec=grid_spec,
        out_shape=jax.ShapeDtypeStruct((n_out, ns), dt),
    )(perm, *([src3] * _G))


def _gather_scale(src, perm, vals, n_out):
    H = src.shape[1]
    src3 = src.reshape(src.shape[0], 1, H)

    def body(perm_ref, *refs):
        in_refs = refs[:_G]
        v_ref = refs[_G]
        o_ref = refs[_G + 1]
        for g in range(_G):
            o_ref[g:g + 1, :] = in_refs[g][0] * v_ref[g:g + 1, :]

    def mk_map(g):
        return lambda i, pref: (pref[i * _G + g], 0, 0)

    grid_spec = pltpu.PrefetchScalarGridSpec(
        num_scalar_prefetch=1,
        grid=(n_out // _G,),
        in_specs=[pl.BlockSpec((1, 1, H), mk_map(g)) for g in range(_G)]
        + [pl.BlockSpec((_G, 1), lambda i, pref: (i, 0))],
        out_specs=pl.BlockSpec((_G, H), lambda i, pref: (i, 0)),
    )
    return pl.pallas_call(
        body,
        grid_spec=grid_spec,
        out_shape=jax.ShapeDtypeStruct((n_out, H), _F32),
    )(perm, *([src3] * _G), vals)


def _unpool(res, hsrc, ip):
    n, H = res.shape
    hsrc3 = hsrc.reshape(hsrc.shape[0], 1, H)

    def body(ip_ref, *refs):
        in_refs = refs[:_G]
        res_ref = refs[_G]
        o_ref = refs[_G + 1]
        i = pl.program_id(0)
        for g in range(_G):
            sel = ip_ref[i * _G + g] >= 0
            o_ref[g:g + 1, :] = res_ref[g:g + 1, :] + jnp.where(
                sel, in_refs[g][0], 0.0)

    def mk_map(g):
        return lambda i, ipref: (jnp.maximum(ipref[i * _G + g], 0), 0, 0)

    grid_spec = pltpu.PrefetchScalarGridSpec(
        num_scalar_prefetch=1,
        grid=(n // _G,),
        in_specs=[pl.BlockSpec((1, 1, H), mk_map(g)) for g in range(_G)]
        + [pl.BlockSpec((_G, H), lambda i, ipref: (i, 0))],
        out_specs=pl.BlockSpec((_G, H), lambda i, ipref: (i, 0)),
    )
    return pl.pallas_call(
        body,
        grid_spec=grid_spec,
        out_shape=jax.ShapeDtypeStruct((n, H), _F32),
    )(ip, *([hsrc3] * _G), res)


def kernel(x, edge_index, batch, enc_W, enc_b, down_W, down_b, pool_w,
           up_W, up_b, up_Wl, up_bl):
    N, Din = x.shape
    H = enc_W.shape[1]
    depth = pool_w.shape[0]

    reals = [N]
    for _ in range(depth):
        reals.append(-(-reals[-1] // 2))

    def _pad_size(r):
        p = _rup(r, 128)
        while p >= 1024 and _bdiv(p, 512) < 256:
            p += 128
        return p

    pads = [_pad_size(r) for r in reals]

    M0 = _mm(enc_W, down_W[0])
    m0 = _mm(enc_b.reshape(1, H), down_W[0])
    xp = jnp.pad(x, ((0, pads[0] - N), (0, 0)))
    XW0 = _mm(xp, M0, bias=m0)

    A = jnp.zeros((pads[0], pads[0]), _F32).at[
        edge_index[0], edge_index[1]].add(1.0)
    AT = _transpose(A)
    cs0, diag0 = _colsum_diag(A, with_diag=True)
    h = _conv(A, XW0, cs0, diag0, down_b[0].reshape(1, H),
              relu=True, real=N, lvl0=True)

    xs = [h]
    As = [(A, cs0, diag0)]
    perms = []
    A_cur, AT_cur = A, AT
    for i in range(1, depth + 1):
        prev_real, prev_pad = reals[i - 1], pads[i - 1]
        k, np_i = reals[i], pads[i]
        w = pool_w[i - 1].reshape(H, 1)
        score = _rowdot(h, w, mode="score", real=prev_real)
        vals, perm = jax.lax.top_k(score[:prev_real, 0], k)
        perm = perm.astype(jnp.int32)
        perm_pad = jnp.pad(perm, (0, np_i - k))
        vals_pad = jnp.pad(vals, (0, np_i - k)).reshape(np_i, 1)
        hp = _gather_scale(h, perm_pad, vals_pad, np_i)

        dt = jnp.bfloat16 if i == 1 else _F32
        R = _gather_diag1(A_cur, perm_pad, k, np_i, dt)
        Rt = _gather_diag1(AT_cur, perm_pad, k, np_i, dt)
        C = _aug_mm(R, Rt)
        cs, _ = _colsum_diag(C, with_diag=False)
        XW = _mm(hp, down_W[i])
        h = _conv(C, XW, cs, None, down_b[i].reshape(1, H),
                  relu=True, real=k, lvl0=False)
        perms.append(perm)
        if i < depth:
            xs.append(h)
            As.append((C, cs, None))
            AT_cur = _transpose(C)
            A_cur = C

    for ui in range(depth):
        j = depth - 1 - ui
        res = xs[j]
        A_j, cs_j, diag_j = As[j]
        perm = perms[j]
        k_next = reals[j + 1]
        ip = jnp.full((pads[j],), -1, jnp.int32).at[perm].set(
            jnp.arange(k_next, dtype=jnp.int32))
        hsum = _unpool(res, h, ip)
        if ui < depth - 1:
            XW = _mm(hsum, up_W[ui])
            h = _conv(A_j, XW, cs_j, None, up_b[ui].reshape(1, H),
                      relu=True, real=reals[j], lvl0=False)
        else:
            y = _rowdot(hsum, up_Wl.reshape(H, 1), mode="plain", real=N)
            out = _conv(A_j, y, cs_j, diag_j, up_bl.reshape(1, 1),
                        relu=False, real=N, lvl0=True)
            return out[:N]


# speedup vs baseline: 1.0151x; 1.0151x over previous
import functools

import jax
import jax.numpy as jnp
from jax.experimental import pallas as pl
from jax.experimental.pallas import tpu as pltpu

_F32 = jnp.float32


def _rup(v, m):
    return -(-v // m) * m


def _bdiv(total, cap, step=128):
    if total < step:
        return total
    best = None
    d = step
    while d <= total:
        if total % d == 0 and d <= cap:
            best = d
        d += step
    assert best is not None, (total, cap, step)
    return best


def _mm(a, b, bias=None):
    M, K = a.shape
    K2, H = b.shape
    assert K == K2
    bm = _bdiv(M, 1024, step=8)
    have_bias = bias is not None

    def body(*refs):
        if have_bias:
            a_ref, b_ref, bias_ref, o_ref = refs
        else:
            a_ref, b_ref, o_ref = refs
        acc = jnp.dot(a_ref[...], b_ref[...], preferred_element_type=_F32)
        if have_bias:
            acc = acc + bias_ref[...]
        o_ref[...] = acc

    in_specs = [
        pl.BlockSpec((bm, K), lambda i: (i, 0)),
        pl.BlockSpec((K, H), lambda i: (0, 0)),
    ]
    args = [a, b]
    if have_bias:
        in_specs.append(pl.BlockSpec((1, H), lambda i: (0, 0)))
        args.append(bias)
    return pl.pallas_call(
        body,
        grid=(M // bm,),
        in_specs=in_specs,
        out_specs=pl.BlockSpec((bm, H), lambda i: (i, 0)),
        out_shape=jax.ShapeDtypeStruct((M, H), _F32),
    )(*args)


def _transpose(a):
    n, m = a.shape
    bi = _bdiv(n, 512)
    bj = _bdiv(m, 512)

    def body(a_ref, o_ref):
        o_ref[...] = a_ref[...].T

    return pl.pallas_call(
        body,
        grid=(n // bi, m // bj),
        in_specs=[pl.BlockSpec((bi, bj), lambda i, j: (i, j))],
        out_specs=pl.BlockSpec((bj, bi), lambda i, j: (j, i)),
        out_shape=jax.ShapeDtypeStruct((m, n), a.dtype),
    )(a)


def _aug_mm(a, b):
    M, K = a.shape
    N2, K2 = b.shape
    assert K == K2
    bm = _bdiv(M, 512)
    bn = _bdiv(N2, 512)
    bk = _bdiv(K, 2048)
    nk = K // bk

    def body(a_ref, b_ref, o_ref, acc):
        i = pl.program_id(0)
        j = pl.program_id(1)
        k = pl.program_id(2)

        @pl.when(k == 0)
        def _():
            acc[...] = jnp.zeros_like(acc)

        acc[...] += jax.lax.dot_general(
            a_ref[...], b_ref[...], (((1,), (1,)), ((), ())),
            preferred_element_type=_F32)

        @pl.when(k == nk - 1)
        def _():
            r = jax.lax.broadcasted_iota(jnp.int32, (bm, bn), 0) + i * bm
            c = jax.lax.broadcasted_iota(jnp.int32, (bm, bn), 1) + j * bn
            o_ref[...] = jnp.where(r == c, 0.0, acc[...])

    return pl.pallas_call(
        body,
        grid=(M // bm, N2 // bn, nk),
        in_specs=[
            pl.BlockSpec((bm, bk), lambda i, j, k: (i, k)),
            pl.BlockSpec((bn, bk), lambda i, j, k: (j, k)),
        ],
        out_specs=pl.BlockSpec((bm, bn), lambda i, j, k: (i, j)),
        out_shape=jax.ShapeDtypeStruct((M, N2), _F32),
        scratch_shapes=[pltpu.VMEM((bm, bn), _F32)],
    )(a, b)


def _colsum_diag(a, with_diag):
    n = a.shape[0]
    bn = _bdiv(n, 512)
    bk = _bdiv(n, 512)
    nk = n // bk
    n_out = 2 if with_diag else 1

    def body(a_ref, *refs):
        outs = refs[:n_out]
        accs = refs[n_out:]
        k = pl.program_id(1)

        @pl.when(k == 0)
        def _():
            for acc in accs:
                acc[...] = jnp.zeros_like(acc)

        blk = a_ref[...]
        ones = jnp.ones((bk, 1), _F32)
        accs[0][...] += jax.lax.dot_general(
            blk, ones, (((0,), (0,)), ((), ())), preferred_element_type=_F32,
            precision=jax.lax.Precision.HIGHEST)
        if with_diag:
            j = pl.program_id(0)
            r = jax.lax.broadcasted_iota(jnp.int32, (bk, bn), 0) + k * bk
            c = jax.lax.broadcasted_iota(jnp.int32, (bk, bn), 1) + j * bn
            accs[1][...] += jax.lax.dot_general(
                jnp.where(r == c, blk, 0.0), ones, (((0,), (0,)), ((), ())),
                preferred_element_type=_F32,
                precision=jax.lax.Precision.HIGHEST)

        @pl.when(k == nk - 1)
        def _():
            for o, acc in zip(outs, accs):
                o[...] = acc[...]

    out_shape = [jax.ShapeDtypeStruct((n, 1), _F32)] * n_out
    res = pl.pallas_call(
        body,
        grid=(n // bn, nk),
        in_specs=[pl.BlockSpec((bk, bn), lambda j, k: (k, j))],
        out_specs=[pl.BlockSpec((bn, 1), lambda j, k: (j, 0))] * n_out,
        out_shape=out_shape,
        scratch_shapes=[pltpu.VMEM((bn, 1), _F32)] * n_out,
    )(a)
    if with_diag:
        return res[0], res[1]
    return res[0], None


def _conv(a, xw, dinv, diag, bias, relu, real, lvl0):
    n = a.shape[0]
    H = xw.shape[1]
    bm = _bdiv(n, 512)
    bk = _bdiv(n, 512)
    nk = n // bk
    dinv_row = dinv.reshape(1, n)

    def body(*refs):
        if lvl0:
            (a_ref, xwk_ref, dck_ref, diagk_ref, drm_ref, b_ref, o_ref,
             acc) = refs
        else:
            (a_ref, xwk_ref, dck_ref, drm_ref, b_ref, o_ref, acc) = refs
            diagk_ref = None
        m = pl.program_id(0)
        k = pl.program_id(1)

        @pl.when(k == 0)
        def _():
            acc[...] = jnp.zeros_like(acc)

        blk = a_ref[...]
        r = jax.lax.broadcasted_iota(jnp.int32, (bk, bm), 0) + k * bk
        c = jax.lax.broadcasted_iota(jnp.int32, (bk, bm), 1) + m * bm
        if lvl0:
            dvals = diagk_ref[...]
            a2d = jnp.where(dvals > 0, dvals, 2.0)
        else:
            a2d = 2.0
        a2 = jnp.where(r == c, a2d, blk)
        ahat = (dck_ref[...] * a2) * drm_ref[...]
        acc[...] += jax.lax.dot_general(
            ahat, xwk_ref[...], (((0,), (0,)), ((), ())),
            preferred_element_type=_F32)

        @pl.when(k == nk - 1)
        def _():
            o = acc[...] + b_ref[...]
            if relu:
                o = jnp.maximum(o, 0.0)
            rows = jax.lax.broadcasted_iota(jnp.int32, (bm, 1), 0) + m * bm
            o_ref[...] = jnp.where(rows < real, o, 0.0)

    in_specs = [pl.BlockSpec((bk, bm), lambda m, k: (k, m)),
                pl.BlockSpec((bk, H), lambda m, k: (k, 0)),
                pl.BlockSpec((bk, 1), lambda m, k: (k, 0))]
    args = [a, xw, dinv]
    if lvl0:
        in_specs.append(pl.BlockSpec((bk, 1), lambda m, k: (k, 0)))
        args.append(diag)
    in_specs.append(pl.BlockSpec((1, bm), lambda m, k: (0, m)))
    args.append(dinv_row)
    in_specs.append(pl.BlockSpec((1, H), lambda m, k: (0, 0)))
    args.append(bias)

    return pl.pallas_call(
        body,
        grid=(n // bm, nk),
        in_specs=in_specs,
        out_specs=pl.BlockSpec((bm, H), lambda m, k: (m, 0)),
        out_shape=jax.ShapeDtypeStruct((n, H), _F32),
        scratch_shapes=[pltpu.VMEM((bm, H), _F32)],
    )(*args)


def _rowdot(h, w, mode, real):
    n, H = h.shape
    bm = _bdiv(n, 1024, step=8)

    def body(h_ref, w_ref, o_ref):
        wv = w_ref[...]
        d = jnp.dot(h_ref[...], wv, preferred_element_type=_F32)
        if mode == "score":
            d = jnp.tanh(d * jax.lax.rsqrt(jnp.sum(wv * wv)))
            pad_val = -2.0
        else:
            pad_val = 0.0
        m = pl.program_id(0)
        rows = jax.lax.broadcasted_iota(jnp.int32, (bm, 1), 0) + m * bm
        o_ref[...] = jnp.where(rows < real, d, pad_val)

    return pl.pallas_call(
        body,
        grid=(n // bm,),
        in_specs=[pl.BlockSpec((bm, H), lambda m: (m, 0)),
                  pl.BlockSpec((H, 1), lambda m: (0, 0))],
        out_specs=pl.BlockSpec((bm, 1), lambda m: (m, 0)),
        out_shape=jax.ShapeDtypeStruct((n, 1), _F32),
    )(h, w)


_G = 8


def _gather_diag1(src, perm, real, n_out, dt):
    ns = src.shape[1]
    src3 = src.reshape(src.shape[0], 1, ns)

    def body(perm_ref, *refs):
        in_refs = refs[:_G]
        o_ref = refs[_G]
        i = pl.program_id(0)
        cid = jax.lax.broadcasted_iota(jnp.int32, (1, ns), 1)
        for g in range(_G):
            j = i * _G + g
            col = perm_ref[j]
            row = jnp.where(cid == col, 1.0, in_refs[g][0])
            row = jnp.where(j < real, row, 0.0)
            o_ref[g:g + 1, :] = row.astype(dt)

    def mk_map(g):
        return lambda i, pref: (pref[i * _G + g], 0, 0)

    grid_spec = pltpu.PrefetchScalarGridSpec(
        num_scalar_prefetch=1,
        grid=(n_out // _G,),
        in_specs=[pl.BlockSpec((1, 1, ns), mk_map(g)) for g in range(_G)],
        out_specs=pl.BlockSpec((_G, ns), lambda i, pref: (i, 0)),
    )
    return pl.pallas_call(
        body,
        grid_spec=grid_spec,
        out_shape=jax.ShapeDtypeStruct((n_out, ns), dt),
    )(perm, *([src3] * _G))


def _gather_scale(src, perm, vals, n_out):
    H = src.shape[1]
    src3 = src.reshape(src.shape[0], 1, H)

    def body(perm_ref, *refs):
        in_refs = refs[:_G]
        v_ref = refs[_G]
        o_ref = refs[_G + 1]
        for g in range(_G):
            o_ref[g:g + 1, :] = in_refs[g][0] * v_ref[g:g + 1, :]

    def mk_map(g):
        return lambda i, pref: (pref[i * _G + g], 0, 0)

    grid_spec = pltpu.PrefetchScalarGridSpec(
        num_scalar_prefetch=1,
        grid=(n_out // _G,),
        in_specs=[pl.BlockSpec((1, 1, H), mk_map(g)) for g in range(_G)]
        + [pl.BlockSpec((_G, 1), lambda i, pref: (i, 0))],
        out_specs=pl.BlockSpec((_G, H), lambda i, pref: (i, 0)),
    )
    return pl.pallas_call(
        body,
        grid_spec=grid_spec,
        out_shape=jax.ShapeDtypeStruct((n_out, H), _F32),
    )(perm, *([src3] * _G), vals)


def _unpool(res, hsrc, ip):
    n, H = res.shape
    hsrc3 = hsrc.reshape(hsrc.shape[0], 1, H)

    def body(ip_ref, *refs):
        in_refs = refs[:_G]
        res_ref = refs[_G]
        o_ref = refs[_G + 1]
        i = pl.program_id(0)
        for g in range(_G):
            sel = ip_ref[i * _G + g] >= 0
            o_ref[g:g + 1, :] = res_ref[g:g + 1, :] + jnp.where(
                sel, in_refs[g][0], 0.0)

    def mk_map(g):
        return lambda i, ipref: (jnp.maximum(ipref[i * _G + g], 0), 0, 0)

    grid_spec = pltpu.PrefetchScalarGridSpec(
        num_scalar_prefetch=1,
        grid=(n // _G,),
        in_specs=[pl.BlockSpec((1, 1, H), mk_map(g)) for g in range(_G)]
        + [pl.BlockSpec((_G, H), lambda i, ipref: (i, 0))],
        out_specs=pl.BlockSpec((_G, H), lambda i, ipref: (i, 0)),
    )
    return pl.pallas_call(
        body,
        grid_spec=grid_spec,
        out_shape=jax.ShapeDtypeStruct((n, H), _F32),
    )(ip, *([hsrc3] * _G), res)


def kernel(x, edge_index, batch, enc_W, enc_b, down_W, down_b, pool_w,
           up_W, up_b, up_Wl, up_bl):
    N, Din = x.shape
    H = enc_W.shape[1]
    depth = pool_w.shape[0]

    reals = [N]
    for _ in range(depth):
        reals.append(-(-reals[-1] // 2))

    def _pad_size(r):
        p = _rup(r, 128)
        while p >= 1024 and _bdiv(p, 512) < 256:
            p += 128
        return p

    pads = [_pad_size(r) for r in reals]

    xp = jnp.pad(x, ((0, pads[0] - N), (0, 0)))
    henc = _mm(xp, enc_W, bias=enc_b.reshape(1, H))
    XW0 = _mm(henc, down_W[0])

    A = jnp.zeros((pads[0], pads[0]), _F32).at[
        edge_index[0], edge_index[1]].add(1.0)
    AT = _transpose(A)
    cs0, diag0 = _colsum_diag(A, with_diag=True)
    a2d0 = jnp.where(diag0 > 0, diag0, 2.0)
    deg0 = cs0 - diag0 + a2d0
    dinv0 = jnp.where(deg0 > 0, 1.0 / jnp.sqrt(deg0), 0.0)
    h = _conv(A, XW0, dinv0, diag0, down_b[0].reshape(1, H),
              relu=True, real=N, lvl0=True)

    xs = [h]
    As = [(A, dinv0, diag0)]
    perms = []
    A_cur, AT_cur = A, AT
    for i in range(1, depth + 1):
        prev_real, prev_pad = reals[i - 1], pads[i - 1]
        k, np_i = reals[i], pads[i]
        w = pool_w[i - 1].reshape(H, 1)
        hw = _rowdot(h, w, mode="plain", real=prev_real)
        score = jnp.tanh(hw[:prev_real, 0] / jnp.linalg.norm(pool_w[i - 1]))
        vals, perm = jax.lax.top_k(score, k)
        perm = perm.astype(jnp.int32)
        perm_pad = jnp.pad(perm, (0, np_i - k))
        vals_pad = jnp.pad(vals, (0, np_i - k)).reshape(np_i, 1)
        hp = _gather_scale(h, perm_pad, vals_pad, np_i)

        dt = jnp.bfloat16 if i == 1 else _F32
        R = _gather_diag1(A_cur, perm_pad, k, np_i, dt)
        Rt = _gather_diag1(AT_cur, perm_pad, k, np_i, dt)
        C = _aug_mm(R, Rt)
        cs, _ = _colsum_diag(C, with_diag=False)
        dinv = jnp.where(cs + 2.0 > 0, 1.0 / jnp.sqrt(cs + 2.0), 0.0)
        XW = _mm(hp, down_W[i])
        h = _conv(C, XW, dinv, None, down_b[i].reshape(1, H),
                  relu=True, real=k, lvl0=False)
        perms.append(perm)
        if i < depth:
            xs.append(h)
            As.append((C, dinv, None))
            AT_cur = _transpose(C)
            A_cur = C

    for ui in range(depth):
        j = depth - 1 - ui
        res = xs[j]
        A_j, dinv_j, diag_j = As[j]
        perm = perms[j]
        k_next = reals[j + 1]
        ip = jnp.full((pads[j],), -1, jnp.int32).at[perm].set(
            jnp.arange(k_next, dtype=jnp.int32))
        hsum = _unpool(res, h, ip)
        if ui < depth - 1:
            XW = _mm(hsum, up_W[ui])
            h = _conv(A_j, XW, dinv_j, None, up_b[ui].reshape(1, H),
                      relu=True, real=reals[j], lvl0=False)
        else:
            y = _rowdot(hsum, up_Wl.reshape(H, 1), mode="plain", real=N)
            out = _conv(A_j, y, dinv_j, diag_j, up_bl.reshape(1, 1),
                        relu=False, real=N, lvl0=True)
            return out[:N]


# speedup vs baseline: 1.2267x; 1.2084x over previous
import functools

import jax
import jax.numpy as jnp
from jax.experimental import pallas as pl
from jax.experimental.pallas import tpu as pltpu

_F32 = jnp.float32


def _rup(v, m):
    return -(-v // m) * m


def _bdiv(total, cap, step=128):
    if total < step:
        return total
    best = None
    d = step
    while d <= total:
        if total % d == 0 and d <= cap:
            best = d
        d += step
    assert best is not None, (total, cap, step)
    return best


def _mm(a, b, bias=None):
    M, K = a.shape
    K2, H = b.shape
    assert K == K2
    bm = _bdiv(M, 1024, step=8)
    have_bias = bias is not None

    def body(*refs):
        if have_bias:
            a_ref, b_ref, bias_ref, o_ref = refs
        else:
            a_ref, b_ref, o_ref = refs
        acc = jnp.dot(a_ref[...], b_ref[...], preferred_element_type=_F32)
        if have_bias:
            acc = acc + bias_ref[...]
        o_ref[...] = acc

    in_specs = [
        pl.BlockSpec((bm, K), lambda i: (i, 0)),
        pl.BlockSpec((K, H), lambda i: (0, 0)),
    ]
    args = [a, b]
    if have_bias:
        in_specs.append(pl.BlockSpec((1, H), lambda i: (0, 0)))
        args.append(bias)
    return pl.pallas_call(
        body,
        grid=(M // bm,),
        in_specs=in_specs,
        out_specs=pl.BlockSpec((bm, H), lambda i: (i, 0)),
        out_shape=jax.ShapeDtypeStruct((M, H), _F32),
    )(*args)


def _transpose(a):
    n, m = a.shape
    bi = _bdiv(n, 512)
    bj = _bdiv(m, 512)

    def body(a_ref, o_ref):
        o_ref[...] = a_ref[...].T

    return pl.pallas_call(
        body,
        grid=(n // bi, m // bj),
        in_specs=[pl.BlockSpec((bi, bj), lambda i, j: (i, j))],
        out_specs=pl.BlockSpec((bj, bi), lambda i, j: (j, i)),
        out_shape=jax.ShapeDtypeStruct((m, n), a.dtype),
    )(a)


def _aug_mm(a, b):
    M, K = a.shape
    N2, K2 = b.shape
    assert K == K2
    bm = _bdiv(M, 1024)
    bn = _bdiv(N2, 1024)
    bk = _bdiv(K, 2048)
    nk = K // bk

    def body(a_ref, b_ref, o_ref, acc):
        i = pl.program_id(0)
        j = pl.program_id(1)
        k = pl.program_id(2)

        @pl.when(k == 0)
        def _():
            acc[...] = jnp.zeros_like(acc)

        acc[...] += jax.lax.dot_general(
            a_ref[...], b_ref[...], (((1,), (1,)), ((), ())),
            preferred_element_type=_F32)

        @pl.when(k == nk - 1)
        def _():
            r = jax.lax.broadcasted_iota(jnp.int32, (bm, bn), 0) + i * bm
            c = jax.lax.broadcasted_iota(jnp.int32, (bm, bn), 1) + j * bn
            o_ref[...] = jnp.where(r == c, 0.0, acc[...])

    return pl.pallas_call(
        body,
        grid=(M // bm, N2 // bn, nk),
        in_specs=[
            pl.BlockSpec((bm, bk), lambda i, j, k: (i, k)),
            pl.BlockSpec((bn, bk), lambda i, j, k: (j, k)),
        ],
        out_specs=pl.BlockSpec((bm, bn), lambda i, j, k: (i, j)),
        out_shape=jax.ShapeDtypeStruct((M, N2), _F32),
        scratch_shapes=[pltpu.VMEM((bm, bn), _F32)],
    )(a, b)


def _colsum_diag(a, with_diag):
    n = a.shape[0]
    bn = _bdiv(n, 512)
    bk = _bdiv(n, 512)
    nk = n // bk
    n_out = 2 if with_diag else 1

    def body(a_ref, *refs):
        outs = refs[:n_out]
        accs = refs[n_out:]
        k = pl.program_id(1)

        @pl.when(k == 0)
        def _():
            for acc in accs:
                acc[...] = jnp.zeros_like(acc)

        blk = a_ref[...]
        ones = jnp.ones((bk, 1), _F32)
        accs[0][...] += jax.lax.dot_general(
            blk, ones, (((0,), (0,)), ((), ())), preferred_element_type=_F32,
            precision=jax.lax.Precision.HIGHEST)
        if with_diag:
            j = pl.program_id(0)
            r = jax.lax.broadcasted_iota(jnp.int32, (bk, bn), 0) + k * bk
            c = jax.lax.broadcasted_iota(jnp.int32, (bk, bn), 1) + j * bn
            accs[1][...] += jax.lax.dot_general(
                jnp.where(r == c, blk, 0.0), ones, (((0,), (0,)), ((), ())),
                preferred_element_type=_F32,
                precision=jax.lax.Precision.HIGHEST)

        @pl.when(k == nk - 1)
        def _():
            for o, acc in zip(outs, accs):
                o[...] = acc[...]

    out_shape = [jax.ShapeDtypeStruct((n, 1), _F32)] * n_out
    res = pl.pallas_call(
        body,
        grid=(n // bn, nk),
        in_specs=[pl.BlockSpec((bk, bn), lambda j, k: (k, j))],
        out_specs=[pl.BlockSpec((bn, 1), lambda j, k: (j, 0))] * n_out,
        out_shape=out_shape,
        scratch_shapes=[pltpu.VMEM((bn, 1), _F32)] * n_out,
    )(a)
    if with_diag:
        return res[0], res[1]
    return res[0], None


def _conv(a, xw, dinv, diag, bias, relu, real, lvl0):
    n = a.shape[0]
    H = xw.shape[1]
    bm = _bdiv(n, 512)
    bk = _bdiv(n, 512)
    nk = n // bk
    dinv_row = dinv.reshape(1, n)

    def body(*refs):
        if lvl0:
            (a_ref, xwk_ref, dck_ref, diagk_ref, drm_ref, b_ref, o_ref,
             acc) = refs
        else:
            (a_ref, xwk_ref, dck_ref, drm_ref, b_ref, o_ref, acc) = refs
            diagk_ref = None
        m = pl.program_id(0)
        k = pl.program_id(1)

        @pl.when(k == 0)
        def _():
            acc[...] = jnp.zeros_like(acc)

        blk = a_ref[...]
        r = jax.lax.broadcasted_iota(jnp.int32, (bk, bm), 0) + k * bk
        c = jax.lax.broadcasted_iota(jnp.int32, (bk, bm), 1) + m * bm
        if lvl0:
            dvals = diagk_ref[...]
            a2d = jnp.where(dvals > 0, dvals, 2.0)
        else:
            a2d = 2.0
        a2 = jnp.where(r == c, a2d, blk)
        ahat = (dck_ref[...] * a2) * drm_ref[...]
        acc[...] += jax.lax.dot_general(
            ahat, xwk_ref[...], (((0,), (0,)), ((), ())),
            preferred_element_type=_F32)

        @pl.when(k == nk - 1)
        def _():
            o = acc[...] + b_ref[...]
            if relu:
                o = jnp.maximum(o, 0.0)
            rows = jax.lax.broadcasted_iota(jnp.int32, (bm, 1), 0) + m * bm
            o_ref[...] = jnp.where(rows < real, o, 0.0)

    in_specs = [pl.BlockSpec((bk, bm), lambda m, k: (k, m)),
                pl.BlockSpec((bk, H), lambda m, k: (k, 0)),
                pl.BlockSpec((bk, 1), lambda m, k: (k, 0))]
    args = [a, xw, dinv]
    if lvl0:
        in_specs.append(pl.BlockSpec((bk, 1), lambda m, k: (k, 0)))
        args.append(diag)
    in_specs.append(pl.BlockSpec((1, bm), lambda m, k: (0, m)))
    args.append(dinv_row)
    in_specs.append(pl.BlockSpec((1, H), lambda m, k: (0, 0)))
    args.append(bias)

    return pl.pallas_call(
        body,
        grid=(n // bm, nk),
        in_specs=in_specs,
        out_specs=pl.BlockSpec((bm, H), lambda m, k: (m, 0)),
        out_shape=jax.ShapeDtypeStruct((n, H), _F32),
        scratch_shapes=[pltpu.VMEM((bm, H), _F32)],
    )(*args)


def _rowdot(h, w, mode, real):
    n, H = h.shape
    bm = _bdiv(n, 1024, step=8)

    def body(h_ref, w_ref, o_ref):
        wv = w_ref[...]
        d = jnp.dot(h_ref[...], wv, preferred_element_type=_F32)
        if mode == "score":
            d = jnp.tanh(d * jax.lax.rsqrt(jnp.sum(wv * wv)))
            pad_val = -2.0
        else:
            pad_val = 0.0
        m = pl.program_id(0)
        rows = jax.lax.broadcasted_iota(jnp.int32, (bm, 1), 0) + m * bm
        o_ref[...] = jnp.where(rows < real, d, pad_val)

    return pl.pallas_call(
        body,
        grid=(n // bm,),
        in_specs=[pl.BlockSpec((bm, H), lambda m: (m, 0)),
                  pl.BlockSpec((H, 1), lambda m: (0, 0))],
        out_specs=pl.BlockSpec((bm, 1), lambda m: (m, 0)),
        out_shape=jax.ShapeDtypeStruct((n, 1), _F32),
    )(h, w)


_G = 16


def _gather_diag1(src, perm, real, n_out, dt):
    ns = src.shape[1]
    src3 = src.reshape(src.shape[0], 1, ns)

    def body(perm_ref, *refs):
        in_refs = refs[:_G]
        o_ref = refs[_G]
        i = pl.program_id(0)
        cid = jax.lax.broadcasted_iota(jnp.int32, (1, ns), 1)
        for g in range(_G):
            j = i * _G + g
            col = perm_ref[j]
            row = jnp.where(cid == col, 1.0, in_refs[g][0])
            row = jnp.where(j < real, row, 0.0)
            o_ref[g:g + 1, :] = row.astype(dt)

    def mk_map(g):
        return lambda i, pref: (pref[i * _G + g], 0, 0)

    grid_spec = pltpu.PrefetchScalarGridSpec(
        num_scalar_prefetch=1,
        grid=(n_out // _G,),
        in_specs=[pl.BlockSpec((1, 1, ns), mk_map(g)) for g in range(_G)],
        out_specs=pl.BlockSpec((_G, ns), lambda i, pref: (i, 0)),
    )
    return pl.pallas_call(
        body,
        grid_spec=grid_spec,
        out_shape=jax.ShapeDtypeStruct((n_out, ns), dt),
    )(perm, *([src3] * _G))


def _gather_scale(src, perm, vals, n_out):
    H = src.shape[1]
    src3 = src.reshape(src.shape[0], 1, H)

    def body(perm_ref, *refs):
        in_refs = refs[:_G]
        v_ref = refs[_G]
        o_ref = refs[_G + 1]
        for g in range(_G):
            o_ref[g:g + 1, :] = in_refs[g][0] * v_ref[g:g + 1, :]

    def mk_map(g):
        return lambda i, pref: (pref[i * _G + g], 0, 0)

    grid_spec = pltpu.PrefetchScalarGridSpec(
        num_scalar_prefetch=1,
        grid=(n_out // _G,),
        in_specs=[pl.BlockSpec((1, 1, H), mk_map(g)) for g in range(_G)]
        + [pl.BlockSpec((_G, 1), lambda i, pref: (i, 0))],
        out_specs=pl.BlockSpec((_G, H), lambda i, pref: (i, 0)),
    )
    return pl.pallas_call(
        body,
        grid_spec=grid_spec,
        out_shape=jax.ShapeDtypeStruct((n_out, H), _F32),
    )(perm, *([src3] * _G), vals)


def _unpool(res, hsrc, ip):
    n, H = res.shape
    hsrc3 = hsrc.reshape(hsrc.shape[0], 1, H)

    def body(ip_ref, *refs):
        in_refs = refs[:_G]
        res_ref = refs[_G]
        o_ref = refs[_G + 1]
        i = pl.program_id(0)
        for g in range(_G):
            sel = ip_ref[i * _G + g] >= 0
            o_ref[g:g + 1, :] = res_ref[g:g + 1, :] + jnp.where(
                sel, in_refs[g][0], 0.0)

    def mk_map(g):
        return lambda i, ipref: (jnp.maximum(ipref[i * _G + g], 0), 0, 0)

    grid_spec = pltpu.PrefetchScalarGridSpec(
        num_scalar_prefetch=1,
        grid=(n // _G,),
        in_specs=[pl.BlockSpec((1, 1, H), mk_map(g)) for g in range(_G)]
        + [pl.BlockSpec((_G, H), lambda i, ipref: (i, 0))],
        out_specs=pl.BlockSpec((_G, H), lambda i, ipref: (i, 0)),
    )
    return pl.pallas_call(
        body,
        grid_spec=grid_spec,
        out_shape=jax.ShapeDtypeStruct((n, H), _F32),
    )(ip, *([hsrc3] * _G), res)


def kernel(x, edge_index, batch, enc_W, enc_b, down_W, down_b, pool_w,
           up_W, up_b, up_Wl, up_bl):
    N, Din = x.shape
    H = enc_W.shape[1]
    depth = pool_w.shape[0]

    reals = [N]
    for _ in range(depth):
        reals.append(-(-reals[-1] // 2))

    def _pad_size(r):
        p = _rup(r, 128)
        while p >= 1024 and _bdiv(p, 512) < 256:
            p += 128
        return p

    pads = [_pad_size(r) for r in reals]

    xp = jnp.pad(x, ((0, pads[0] - N), (0, 0)))
    henc = _mm(xp, enc_W, bias=enc_b.reshape(1, H))
    XW0 = _mm(henc, down_W[0])

    A = jnp.zeros((pads[0], pads[0]), _F32).at[
        edge_index[0], edge_index[1]].add(1.0)
    AT = _transpose(A)
    cs0, diag0 = _colsum_diag(A, with_diag=True)
    a2d0 = jnp.where(diag0 > 0, diag0, 2.0)
    deg0 = cs0 - diag0 + a2d0
    dinv0 = jnp.where(deg0 > 0, 1.0 / jnp.sqrt(deg0), 0.0)
    h = _conv(A, XW0, dinv0, diag0, down_b[0].reshape(1, H),
              relu=True, real=N, lvl0=True)

    xs = [h]
    As = [(A, dinv0, diag0)]
    perms = []
    A_cur, AT_cur = A, AT
    for i in range(1, depth + 1):
        prev_real, prev_pad = reals[i - 1], pads[i - 1]
        k, np_i = reals[i], pads[i]
        w = pool_w[i - 1].reshape(H, 1)
        hw = _rowdot(h, w, mode="plain", real=prev_real)
        score = jnp.tanh(hw[:prev_real, 0] / jnp.linalg.norm(pool_w[i - 1]))
        vals, perm = jax.lax.top_k(score, k)
        perm = perm.astype(jnp.int32)
        perm_pad = jnp.pad(perm, (0, np_i - k))
        vals_pad = jnp.pad(vals, (0, np_i - k)).reshape(np_i, 1)
        hp = _gather_scale(h, perm_pad, vals_pad, np_i)

        dt = jnp.bfloat16 if i == 1 else _F32
        R = _gather_diag1(A_cur, perm_pad, k, np_i, dt)
        Rt = _gather_diag1(AT_cur, perm_pad, k, np_i, dt)
        C = _aug_mm(R, Rt)
        cs, _ = _colsum_diag(C, with_diag=False)
        dinv = jnp.where(cs + 2.0 > 0, 1.0 / jnp.sqrt(cs + 2.0), 0.0)
        XW = _mm(hp, down_W[i])
        h = _conv(C, XW, dinv, None, down_b[i].reshape(1, H),
                  relu=True, real=k, lvl0=False)
        perms.append(perm)
        if i < depth:
            xs.append(h)
            As.append((C, dinv, None))
            AT_cur = _transpose(C)
            A_cur = C

    for ui in range(depth):
        j = depth - 1 - ui
        res = xs[j]
        A_j, dinv_j, diag_j = As[j]
        perm = perms[j]
        k_next = reals[j + 1]
        ip = jnp.full((pads[j],), -1, jnp.int32).at[perm].set(
            jnp.arange(k_next, dtype=jnp.int32))
        hsum = _unpool(res, h, ip)
        if ui < depth - 1:
            XW = _mm(hsum, up_W[ui])
            h = _conv(A_j, XW, dinv_j, None, up_b[ui].reshape(1, H),
                      relu=True, real=reals[j], lvl0=False)
        else:
            y = _rowdot(hsum, up_Wl.reshape(H, 1), mode="plain", real=N)
            out = _conv(A_j, y, dinv_j, diag_j, up_bl.reshape(1, 1),
                        relu=False, real=N, lvl0=True)
            return out[:N]


# speedup vs baseline: 1.2931x; 1.0542x over previous
import functools

import jax
import jax.numpy as jnp
from jax.experimental import pallas as pl
from jax.experimental.pallas import tpu as pltpu

_F32 = jnp.float32


def _rup(v, m):
    return -(-v // m) * m


def _bdiv(total, cap, step=128):
    if total < step:
        return total
    best = None
    d = step
    while d <= total:
        if total % d == 0 and d <= cap:
            best = d
        d += step
    assert best is not None, (total, cap, step)
    return best


def _mm(a, b, bias=None):
    M, K = a.shape
    K2, H = b.shape
    assert K == K2
    bm = _bdiv(M, 1024, step=8)
    have_bias = bias is not None

    def body(*refs):
        if have_bias:
            a_ref, b_ref, bias_ref, o_ref = refs
        else:
            a_ref, b_ref, o_ref = refs
        acc = jnp.dot(a_ref[...], b_ref[...], preferred_element_type=_F32)
        if have_bias:
            acc = acc + bias_ref[...]
        o_ref[...] = acc

    in_specs = [
        pl.BlockSpec((bm, K), lambda i: (i, 0)),
        pl.BlockSpec((K, H), lambda i: (0, 0)),
    ]
    args = [a, b]
    if have_bias:
        in_specs.append(pl.BlockSpec((1, H), lambda i: (0, 0)))
        args.append(bias)
    return pl.pallas_call(
        body,
        grid=(M // bm,),
        in_specs=in_specs,
        out_specs=pl.BlockSpec((bm, H), lambda i: (i, 0)),
        out_shape=jax.ShapeDtypeStruct((M, H), _F32),
    )(*args)


def _transpose(a):
    n, m = a.shape
    bi = _bdiv(n, 512)
    bj = _bdiv(m, 512)

    def body(a_ref, o_ref):
        o_ref[...] = a_ref[...].T

    return pl.pallas_call(
        body,
        grid=(n // bi, m // bj),
        in_specs=[pl.BlockSpec((bi, bj), lambda i, j: (i, j))],
        out_specs=pl.BlockSpec((bj, bi), lambda i, j: (j, i)),
        out_shape=jax.ShapeDtypeStruct((m, n), a.dtype),
    )(a)


def _aug_mm(a, b):
    M, K = a.shape
    N2, K2 = b.shape
    assert K == K2
    bm = _bdiv(M, 1024)
    bn = _bdiv(N2, 1024)
    bk = _bdiv(K, 2048)
    nk = K // bk

    def body(a_ref, b_ref, o_ref, acc):
        i = pl.program_id(0)
        j = pl.program_id(1)
        k = pl.program_id(2)

        @pl.when(k == 0)
        def _():
            acc[...] = jnp.zeros_like(acc)

        acc[...] += jax.lax.dot_general(
            a_ref[...], b_ref[...], (((1,), (1,)), ((), ())),
            preferred_element_type=_F32)

        @pl.when(k == nk - 1)
        def _():
            r = jax.lax.broadcasted_iota(jnp.int32, (bm, bn), 0) + i * bm
            c = jax.lax.broadcasted_iota(jnp.int32, (bm, bn), 1) + j * bn
            o_ref[...] = jnp.where(r == c, 0.0, acc[...])

    return pl.pallas_call(
        body,
        grid=(M // bm, N2 // bn, nk),
        in_specs=[
            pl.BlockSpec((bm, bk), lambda i, j, k: (i, k)),
            pl.BlockSpec((bn, bk), lambda i, j, k: (j, k)),
        ],
        out_specs=pl.BlockSpec((bm, bn), lambda i, j, k: (i, j)),
        out_shape=jax.ShapeDtypeStruct((M, N2), _F32),
        scratch_shapes=[pltpu.VMEM((bm, bn), _F32)],
    )(a, b)


def _colsum_diag(a, with_diag):
    n = a.shape[0]
    bn = _bdiv(n, 512)
    bk = _bdiv(n, 512)
    nk = n // bk
    n_out = 2 if with_diag else 1

    def body(a_ref, *refs):
        outs = refs[:n_out]
        accs = refs[n_out:]
        k = pl.program_id(1)

        @pl.when(k == 0)
        def _():
            for acc in accs:
                acc[...] = jnp.zeros_like(acc)

        blk = a_ref[...]
        ones = jnp.ones((bk, 1), _F32)
        accs[0][...] += jax.lax.dot_general(
            blk, ones, (((0,), (0,)), ((), ())), preferred_element_type=_F32,
            precision=jax.lax.Precision.HIGHEST)
        if with_diag:
            j = pl.program_id(0)
            r = jax.lax.broadcasted_iota(jnp.int32, (bk, bn), 0) + k * bk
            c = jax.lax.broadcasted_iota(jnp.int32, (bk, bn), 1) + j * bn
            accs[1][...] += jax.lax.dot_general(
                jnp.where(r == c, blk, 0.0), ones, (((0,), (0,)), ((), ())),
                preferred_element_type=_F32,
                precision=jax.lax.Precision.HIGHEST)

        @pl.when(k == nk - 1)
        def _():
            for o, acc in zip(outs, accs):
                o[...] = acc[...]

    out_shape = [jax.ShapeDtypeStruct((n, 1), _F32)] * n_out
    res = pl.pallas_call(
        body,
        grid=(n // bn, nk),
        in_specs=[pl.BlockSpec((bk, bn), lambda j, k: (k, j))],
        out_specs=[pl.BlockSpec((bn, 1), lambda j, k: (j, 0))] * n_out,
        out_shape=out_shape,
        scratch_shapes=[pltpu.VMEM((bn, 1), _F32)] * n_out,
    )(a)
    if with_diag:
        return res[0], res[1]
    return res[0], None


def _conv(a, xw, dinv, diag, bias, relu, real, lvl0):
    n = a.shape[0]
    H = xw.shape[1]
    bm = _bdiv(n, 512)
    bk = _bdiv(n, 512)
    nk = n // bk
    dinv_row = dinv.reshape(1, n)

    def body(*refs):
        if lvl0:
            (a_ref, xwk_ref, dck_ref, diagk_ref, drm_ref, b_ref, o_ref,
             acc) = refs
        else:
            (a_ref, xwk_ref, dck_ref, drm_ref, b_ref, o_ref, acc) = refs
            diagk_ref = None
        m = pl.program_id(0)
        k = pl.program_id(1)

        @pl.when(k == 0)
        def _():
            acc[...] = jnp.zeros_like(acc)

        blk = a_ref[...]
        r = jax.lax.broadcasted_iota(jnp.int32, (bk, bm), 0) + k * bk
        c = jax.lax.broadcasted_iota(jnp.int32, (bk, bm), 1) + m * bm
        if lvl0:
            dvals = diagk_ref[...]
            a2d = jnp.where(dvals > 0, dvals, 2.0)
        else:
            a2d = 2.0
        a2 = jnp.where(r == c, a2d, blk)
        ahat = (dck_ref[...] * a2) * drm_ref[...]
        acc[...] += jax.lax.dot_general(
            ahat, xwk_ref[...], (((0,), (0,)), ((), ())),
            preferred_element_type=_F32)

        @pl.when(k == nk - 1)
        def _():
            o = acc[...] + b_ref[...]
            if relu:
                o = jnp.maximum(o, 0.0)
            rows = jax.lax.broadcasted_iota(jnp.int32, (bm, 1), 0) + m * bm
            o_ref[...] = jnp.where(rows < real, o, 0.0)

    in_specs = [pl.BlockSpec((bk, bm), lambda m, k: (k, m)),
                pl.BlockSpec((bk, H), lambda m, k: (k, 0)),
                pl.BlockSpec((bk, 1), lambda m, k: (k, 0))]
    args = [a, xw, dinv]
    if lvl0:
        in_specs.append(pl.BlockSpec((bk, 1), lambda m, k: (k, 0)))
        args.append(diag)
    in_specs.append(pl.BlockSpec((1, bm), lambda m, k: (0, m)))
    args.append(dinv_row)
    in_specs.append(pl.BlockSpec((1, H), lambda m, k: (0, 0)))
    args.append(bias)

    return pl.pallas_call(
        body,
        grid=(n // bm, nk),
        in_specs=in_specs,
        out_specs=pl.BlockSpec((bm, H), lambda m, k: (m, 0)),
        out_shape=jax.ShapeDtypeStruct((n, H), _F32),
        scratch_shapes=[pltpu.VMEM((bm, H), _F32)],
    )(*args)


def _rowdot(h, w, mode, real):
    n, H = h.shape
    bm = _bdiv(n, 1024, step=8)

    def body(h_ref, w_ref, o_ref):
        wv = w_ref[...]
        d = jnp.dot(h_ref[...], wv, preferred_element_type=_F32)
        if mode == "score":
            d = jnp.tanh(d * jax.lax.rsqrt(jnp.sum(wv * wv)))
            pad_val = -2.0
        else:
            pad_val = 0.0
        m = pl.program_id(0)
        rows = jax.lax.broadcasted_iota(jnp.int32, (bm, 1), 0) + m * bm
        o_ref[...] = jnp.where(rows < real, d, pad_val)

    return pl.pallas_call(
        body,
        grid=(n // bm,),
        in_specs=[pl.BlockSpec((bm, H), lambda m: (m, 0)),
                  pl.BlockSpec((H, 1), lambda m: (0, 0))],
        out_specs=pl.BlockSpec((bm, 1), lambda m: (m, 0)),
        out_shape=jax.ShapeDtypeStruct((n, 1), _F32),
    )(h, w)


_G = 32


def _gather_diag1(src, perm, real, n_out, dt):
    ns = src.shape[1]
    src3 = src.reshape(src.shape[0], 1, ns)

    def body(perm_ref, *refs):
        in_refs = refs[:_G]
        o_ref = refs[_G]
        i = pl.program_id(0)
        cid = jax.lax.broadcasted_iota(jnp.int32, (1, ns), 1)
        for g in range(_G):
            j = i * _G + g
            col = perm_ref[j]
            row = jnp.where(cid == col, 1.0, in_refs[g][0])
            row = jnp.where(j < real, row, 0.0)
            o_ref[g:g + 1, :] = row.astype(dt)

    def mk_map(g):
        return lambda i, pref: (pref[i * _G + g], 0, 0)

    grid_spec = pltpu.PrefetchScalarGridSpec(
        num_scalar_prefetch=1,
        grid=(n_out // _G,),
        in_specs=[pl.BlockSpec((1, 1, ns), mk_map(g)) for g in range(_G)],
        out_specs=pl.BlockSpec((_G, ns), lambda i, pref: (i, 0)),
    )
    return pl.pallas_call(
        body,
        grid_spec=grid_spec,
        out_shape=jax.ShapeDtypeStruct((n_out, ns), dt),
    )(perm, *([src3] * _G))


def _gather_scale(src, perm, vals, n_out):
    H = src.shape[1]
    src3 = src.reshape(src.shape[0], 1, H)

    def body(perm_ref, *refs):
        in_refs = refs[:_G]
        v_ref = refs[_G]
        o_ref = refs[_G + 1]
        for g in range(_G):
            o_ref[g:g + 1, :] = in_refs[g][0] * v_ref[g:g + 1, :]

    def mk_map(g):
        return lambda i, pref: (pref[i * _G + g], 0, 0)

    grid_spec = pltpu.PrefetchScalarGridSpec(
        num_scalar_prefetch=1,
        grid=(n_out // _G,),
        in_specs=[pl.BlockSpec((1, 1, H), mk_map(g)) for g in range(_G)]
        + [pl.BlockSpec((_G, 1), lambda i, pref: (i, 0))],
        out_specs=pl.BlockSpec((_G, H), lambda i, pref: (i, 0)),
    )
    return pl.pallas_call(
        body,
        grid_spec=grid_spec,
        out_shape=jax.ShapeDtypeStruct((n_out, H), _F32),
    )(perm, *([src3] * _G), vals)


def _unpool(res, hsrc, ip):
    n, H = res.shape
    hsrc3 = hsrc.reshape(hsrc.shape[0], 1, H)

    def body(ip_ref, *refs):
        in_refs = refs[:_G]
        res_ref = refs[_G]
        o_ref = refs[_G + 1]
        i = pl.program_id(0)
        for g in range(_G):
            sel = ip_ref[i * _G + g] >= 0
            o_ref[g:g + 1, :] = res_ref[g:g + 1, :] + jnp.where(
                sel, in_refs[g][0], 0.0)

    def mk_map(g):
        return lambda i, ipref: (jnp.maximum(ipref[i * _G + g], 0), 0, 0)

    grid_spec = pltpu.PrefetchScalarGridSpec(
        num_scalar_prefetch=1,
        grid=(n // _G,),
        in_specs=[pl.BlockSpec((1, 1, H), mk_map(g)) for g in range(_G)]
        + [pl.BlockSpec((_G, H), lambda i, ipref: (i, 0))],
        out_specs=pl.BlockSpec((_G, H), lambda i, ipref: (i, 0)),
    )
    return pl.pallas_call(
        body,
        grid_spec=grid_spec,
        out_shape=jax.ShapeDtypeStruct((n, H), _F32),
    )(ip, *([hsrc3] * _G), res)


def kernel(x, edge_index, batch, enc_W, enc_b, down_W, down_b, pool_w,
           up_W, up_b, up_Wl, up_bl):
    N, Din = x.shape
    H = enc_W.shape[1]
    depth = pool_w.shape[0]

    reals = [N]
    for _ in range(depth):
        reals.append(-(-reals[-1] // 2))

    def _pad_size(r):
        p = _rup(r, 128)
        while p >= 1024 and _bdiv(p, 512) < 256:
            p += 128
        return p

    pads = [_pad_size(r) for r in reals]

    xp = jnp.pad(x, ((0, pads[0] - N), (0, 0)))
    henc = _mm(xp, enc_W, bias=enc_b.reshape(1, H))
    XW0 = _mm(henc, down_W[0])

    A = jnp.zeros((pads[0], pads[0]), _F32).at[
        edge_index[0], edge_index[1]].add(1.0)
    AT = _transpose(A)
    cs0, diag0 = _colsum_diag(A, with_diag=True)
    a2d0 = jnp.where(diag0 > 0, diag0, 2.0)
    deg0 = cs0 - diag0 + a2d0
    dinv0 = jnp.where(deg0 > 0, 1.0 / jnp.sqrt(deg0), 0.0)
    h = _conv(A, XW0, dinv0, diag0, down_b[0].reshape(1, H),
              relu=True, real=N, lvl0=True)

    xs = [h]
    As = [(A, dinv0, diag0)]
    perms = []
    A_cur, AT_cur = A, AT
    for i in range(1, depth + 1):
        prev_real, prev_pad = reals[i - 1], pads[i - 1]
        k, np_i = reals[i], pads[i]
        w = pool_w[i - 1].reshape(H, 1)
        hw = _rowdot(h, w, mode="plain", real=prev_real)
        score = jnp.tanh(hw[:prev_real, 0] / jnp.linalg.norm(pool_w[i - 1]))
        vals, perm = jax.lax.top_k(score, k)
        perm = perm.astype(jnp.int32)
        perm_pad = jnp.pad(perm, (0, np_i - k))
        vals_pad = jnp.pad(vals, (0, np_i - k)).reshape(np_i, 1)
        hp = _gather_scale(h, perm_pad, vals_pad, np_i)

        dt = jnp.bfloat16 if i == 1 else _F32
        R = _gather_diag1(A_cur, perm_pad, k, np_i, dt)
        Rt = _gather_diag1(AT_cur, perm_pad, k, np_i, dt)
        C = _aug_mm(R, Rt)
        cs, _ = _colsum_diag(C, with_diag=False)
        dinv = jnp.where(cs + 2.0 > 0, 1.0 / jnp.sqrt(cs + 2.0), 0.0)
        XW = _mm(hp, down_W[i])
        h = _conv(C, XW, dinv, None, down_b[i].reshape(1, H),
                  relu=True, real=k, lvl0=False)
        perms.append(perm)
        if i < depth:
            xs.append(h)
            As.append((C, dinv, None))
            AT_cur = _transpose(C)
            A_cur = C

    for ui in range(depth):
        j = depth - 1 - ui
        res = xs[j]
        A_j, dinv_j, diag_j = As[j]
        perm = perms[j]
        k_next = reals[j + 1]
        ip = jnp.full((pads[j],), -1, jnp.int32).at[perm].set(
            jnp.arange(k_next, dtype=jnp.int32))
        hsum = _unpool(res, h, ip)
        if ui < depth - 1:
            XW = _mm(hsum, up_W[ui])
            h = _conv(A_j, XW, dinv_j, None, up_b[ui].reshape(1, H),
                      relu=True, real=reals[j], lvl0=False)
        else:
            y = _rowdot(hsum, up_Wl.reshape(H, 1), mode="plain", real=N)
            out = _conv(A_j, y, dinv_j, diag_j, up_bl.reshape(1, 1),
                        relu=False, real=N, lvl0=True)
            return out[:N]


# speedup vs baseline: 1.2934x; 1.0002x over previous
import jax
import jax.numpy as jnp
from jax.experimental import pallas as pl
from jax.experimental.pallas import tpu as pltpu

_F32 = jnp.float32


def _rup(v, m):
    return -(-v // m) * m


def _bdiv(total, cap, step=128):
    if total < step:
        return total
    best = None
    d = step
    while d <= total:
        if total % d == 0 and d <= cap:
            best = d
        d += step
    assert best is not None, (total, cap, step)
    return best


def _mm(a, b, bias=None):
    M, K = a.shape
    K2, H = b.shape
    assert K == K2
    bm = _bdiv(M, 1024, step=8)
    have_bias = bias is not None

    def body(*refs):
        if have_bias:
            a_ref, b_ref, bias_ref, o_ref = refs
        else:
            a_ref, b_ref, o_ref = refs
        acc = jnp.dot(a_ref[...], b_ref[...], preferred_element_type=_F32)
        if have_bias:
            acc = acc + bias_ref[...]
        o_ref[...] = acc

    in_specs = [
        pl.BlockSpec((bm, K), lambda i: (i, 0)),
        pl.BlockSpec((K, H), lambda i: (0, 0)),
    ]
    args = [a, b]
    if have_bias:
        in_specs.append(pl.BlockSpec((1, H), lambda i: (0, 0)))
        args.append(bias)
    return pl.pallas_call(
        body,
        grid=(M // bm,),
        in_specs=in_specs,
        out_specs=pl.BlockSpec((bm, H), lambda i: (i, 0)),
        out_shape=jax.ShapeDtypeStruct((M, H), _F32),
    )(*args)


def _transpose(a):
    n, m = a.shape
    bi = _bdiv(n, 512)
    bj = _bdiv(m, 512)

    def body(a_ref, o_ref):
        o_ref[...] = a_ref[...].T

    return pl.pallas_call(
        body,
        grid=(n // bi, m // bj),
        in_specs=[pl.BlockSpec((bi, bj), lambda i, j: (i, j))],
        out_specs=pl.BlockSpec((bj, bi), lambda i, j: (j, i)),
        out_shape=jax.ShapeDtypeStruct((m, n), a.dtype),
    )(a)


def _aug_mm(a, b):
    M, K = a.shape
    N2, K2 = b.shape
    assert K == K2
    bm = _bdiv(M, 1024)
    bn = _bdiv(N2, 1024)
    bk = _bdiv(K, 2048)
    nk = K // bk

    def body(a_ref, b_ref, o_ref, acc):
        i = pl.program_id(0)
        j = pl.program_id(1)
        k = pl.program_id(2)

        @pl.when(k == 0)
        def _():
            acc[...] = jnp.zeros_like(acc)

        acc[...] += jax.lax.dot_general(
            a_ref[...], b_ref[...], (((1,), (1,)), ((), ())),
            preferred_element_type=_F32)

        @pl.when(k == nk - 1)
        def _():
            r = jax.lax.broadcasted_iota(jnp.int32, (bm, bn), 0) + i * bm
            c = jax.lax.broadcasted_iota(jnp.int32, (bm, bn), 1) + j * bn
            o_ref[...] = jnp.where(r == c, 0.0, acc[...])

    return pl.pallas_call(
        body,
        grid=(M // bm, N2 // bn, nk),
        in_specs=[
            pl.BlockSpec((bm, bk), lambda i, j, k: (i, k)),
            pl.BlockSpec((bn, bk), lambda i, j, k: (j, k)),
        ],
        out_specs=pl.BlockSpec((bm, bn), lambda i, j, k: (i, j)),
        out_shape=jax.ShapeDtypeStruct((M, N2), _F32),
        scratch_shapes=[pltpu.VMEM((bm, bn), _F32)],
    )(a, b)


def _colsum_diag(a, with_diag):
    n = a.shape[0]
    bn = _bdiv(n, 512)
    bk = _bdiv(n, 512)
    nk = n // bk
    n_out = 2 if with_diag else 1

    def body(a_ref, *refs):
        outs = refs[:n_out]
        accs = refs[n_out:]
        k = pl.program_id(1)

        @pl.when(k == 0)
        def _():
            for acc in accs:
                acc[...] = jnp.zeros_like(acc)

        blk = a_ref[...]
        ones = jnp.ones((bk, 1), _F32)
        accs[0][...] += jax.lax.dot_general(
            blk, ones, (((0,), (0,)), ((), ())), preferred_element_type=_F32,
            precision=jax.lax.Precision.HIGHEST)
        if with_diag:
            j = pl.program_id(0)
            r = jax.lax.broadcasted_iota(jnp.int32, (bk, bn), 0) + k * bk
            c = jax.lax.broadcasted_iota(jnp.int32, (bk, bn), 1) + j * bn
            accs[1][...] += jax.lax.dot_general(
                jnp.where(r == c, blk, 0.0), ones, (((0,), (0,)), ((), ())),
                preferred_element_type=_F32,
                precision=jax.lax.Precision.HIGHEST)

        @pl.when(k == nk - 1)
        def _():
            for o, acc in zip(outs, accs):
                o[...] = acc[...]

    out_shape = [jax.ShapeDtypeStruct((n, 1), _F32)] * n_out
    res = pl.pallas_call(
        body,
        grid=(n // bn, nk),
        in_specs=[pl.BlockSpec((bk, bn), lambda j, k: (k, j))],
        out_specs=[pl.BlockSpec((bn, 1), lambda j, k: (j, 0))] * n_out,
        out_shape=out_shape,
        scratch_shapes=[pltpu.VMEM((bn, 1), _F32)] * n_out,
    )(a)
    if with_diag:
        return res[0], res[1]
    return res[0], None


def _conv(a, xw, dinv, diag, bias, relu, real, lvl0):
    n = a.shape[0]
    H = xw.shape[1]
    bm = _bdiv(n, 512)
    bk = _bdiv(n, 512)
    nk = n // bk
    dinv_row = dinv.reshape(1, n)

    def body(*refs):
        if lvl0:
            (a_ref, xwk_ref, dck_ref, diagk_ref, drm_ref, b_ref, o_ref,
             acc) = refs
        else:
            (a_ref, xwk_ref, dck_ref, drm_ref, b_ref, o_ref, acc) = refs
            diagk_ref = None
        m = pl.program_id(0)
        k = pl.program_id(1)

        @pl.when(k == 0)
        def _():
            acc[...] = jnp.zeros_like(acc)

        blk = a_ref[...]
        r = jax.lax.broadcasted_iota(jnp.int32, (bk, bm), 0) + k * bk
        c = jax.lax.broadcasted_iota(jnp.int32, (bk, bm), 1) + m * bm
        if lvl0:
            dvals = diagk_ref[...]
            a2d = jnp.where(dvals > 0, dvals, 2.0)
        else:
            a2d = 2.0
        a2 = jnp.where(r == c, a2d, blk)
        ahat = (dck_ref[...] * a2) * drm_ref[...]
        acc[...] += jax.lax.dot_general(
            ahat, xwk_ref[...], (((0,), (0,)), ((), ())),
            preferred_element_type=_F32)

        @pl.when(k == nk - 1)
        def _():
            o = acc[...] + b_ref[...]
            if relu:
                o = jnp.maximum(o, 0.0)
            rows = jax.lax.broadcasted_iota(jnp.int32, (bm, 1), 0) + m * bm
            o_ref[...] = jnp.where(rows < real, o, 0.0)

    in_specs = [pl.BlockSpec((bk, bm), lambda m, k: (k, m)),
                pl.BlockSpec((bk, H), lambda m, k: (k, 0)),
                pl.BlockSpec((bk, 1), lambda m, k: (k, 0))]
    args = [a, xw, dinv]
    if lvl0:
        in_specs.append(pl.BlockSpec((bk, 1), lambda m, k: (k, 0)))
        args.append(diag)
    in_specs.append(pl.BlockSpec((1, bm), lambda m, k: (0, m)))
    args.append(dinv_row)
    in_specs.append(pl.BlockSpec((1, H), lambda m, k: (0, 0)))
    args.append(bias)

    return pl.pallas_call(
        body,
        grid=(n // bm, nk),
        in_specs=in_specs,
        out_specs=pl.BlockSpec((bm, H), lambda m, k: (m, 0)),
        out_shape=jax.ShapeDtypeStruct((n, H), _F32),
        scratch_shapes=[pltpu.VMEM((bm, H), _F32)],
    )(*args)


def _rowdot(h, w, real):
    n, H = h.shape
    bm = _bdiv(n, 1024, step=8)

    def body(h_ref, w_ref, o_ref):
        d = jnp.dot(h_ref[...], w_ref[...], preferred_element_type=_F32)
        m = pl.program_id(0)
        rows = jax.lax.broadcasted_iota(jnp.int32, (bm, 1), 0) + m * bm
        o_ref[...] = jnp.where(rows < real, d, 0.0)

    return pl.pallas_call(
        body,
        grid=(n // bm,),
        in_specs=[pl.BlockSpec((bm, H), lambda m: (m, 0)),
                  pl.BlockSpec((H, 1), lambda m: (0, 0))],
        out_specs=pl.BlockSpec((bm, 1), lambda m: (m, 0)),
        out_shape=jax.ShapeDtypeStruct((n, 1), _F32),
    )(h, w)


_G = 32


def _gather_diag1(src, perm, real, n_out, dt):
    ns = src.shape[1]
    src3 = src.reshape(src.shape[0], 1, ns)

    def body(perm_ref, *refs):
        in_refs = refs[:_G]
        o_ref = refs[_G]
        i = pl.program_id(0)
        cid = jax.lax.broadcasted_iota(jnp.int32, (1, ns), 1)
        for g in range(_G):
            j = i * _G + g
            col = perm_ref[j]
            row = jnp.where(cid == col, 1.0, in_refs[g][0])
            row = jnp.where(j < real, row, 0.0)
            o_ref[g:g + 1, :] = row.astype(dt)

    def mk_map(g):
        return lambda i, pref: (pref[i * _G + g], 0, 0)

    grid_spec = pltpu.PrefetchScalarGridSpec(
        num_scalar_prefetch=1,
        grid=(n_out // _G,),
        in_specs=[pl.BlockSpec((1, 1, ns), mk_map(g)) for g in range(_G)],
        out_specs=pl.BlockSpec((_G, ns), lambda i, pref: (i, 0)),
    )
    return pl.pallas_call(
        body,
        grid_spec=grid_spec,
        out_shape=jax.ShapeDtypeStruct((n_out, ns), dt),
    )(perm, *([src3] * _G))


def _gather_scale(src, perm, vals, n_out):
    H = src.shape[1]
    src3 = src.reshape(src.shape[0], 1, H)

    def body(perm_ref, *refs):
        in_refs = refs[:_G]
        v_ref = refs[_G]
        o_ref = refs[_G + 1]
        for g in range(_G):
            o_ref[g:g + 1, :] = in_refs[g][0] * v_ref[g:g + 1, :]

    def mk_map(g):
        return lambda i, pref: (pref[i * _G + g], 0, 0)

    grid_spec = pltpu.PrefetchScalarGridSpec(
        num_scalar_prefetch=1,
        grid=(n_out // _G,),
        in_specs=[pl.BlockSpec((1, 1, H), mk_map(g)) for g in range(_G)]
        + [pl.BlockSpec((_G, 1), lambda i, pref: (i, 0))],
        out_specs=pl.BlockSpec((_G, H), lambda i, pref: (i, 0)),
    )
    return pl.pallas_call(
        body,
        grid_spec=grid_spec,
        out_shape=jax.ShapeDtypeStruct((n_out, H), _F32),
    )(perm, *([src3] * _G), vals)


def _unpool(res, hsrc, ip):
    n, H = res.shape
    hsrc3 = hsrc.reshape(hsrc.shape[0], 1, H)

    def body(ip_ref, *refs):
        in_refs = refs[:_G]
        res_ref = refs[_G]
        o_ref = refs[_G + 1]
        i = pl.program_id(0)
        for g in range(_G):
            sel = ip_ref[i * _G + g] >= 0
            o_ref[g:g + 1, :] = res_ref[g:g + 1, :] + jnp.where(
                sel, in_refs[g][0], 0.0)

    def mk_map(g):
        return lambda i, ipref: (jnp.maximum(ipref[i * _G + g], 0), 0, 0)

    grid_spec = pltpu.PrefetchScalarGridSpec(
        num_scalar_prefetch=1,
        grid=(n // _G,),
        in_specs=[pl.BlockSpec((1, 1, H), mk_map(g)) for g in range(_G)]
        + [pl.BlockSpec((_G, H), lambda i, ipref: (i, 0))],
        out_specs=pl.BlockSpec((_G, H), lambda i, ipref: (i, 0)),
    )
    return pl.pallas_call(
        body,
        grid_spec=grid_spec,
        out_shape=jax.ShapeDtypeStruct((n, H), _F32),
    )(ip, *([hsrc3] * _G), res)


def kernel(x, edge_index, batch, enc_W, enc_b, down_W, down_b, pool_w,
           up_W, up_b, up_Wl, up_bl):
    N, Din = x.shape
    H = enc_W.shape[1]
    depth = pool_w.shape[0]

    reals = [N]
    for _ in range(depth):
        reals.append(-(-reals[-1] // 2))

    def _pad_size(r):
        p = _rup(r, 128)
        while p >= 1024 and _bdiv(p, 512) < 256:
            p += 128
        return p

    pads = [_pad_size(r) for r in reals]

    xp = jnp.pad(x, ((0, pads[0] - N), (0, 0)))
    henc = _mm(xp, enc_W, bias=enc_b.reshape(1, H))
    XW0 = _mm(henc, down_W[0])

    A = jnp.zeros((pads[0], pads[0]), _F32).at[
        edge_index[0], edge_index[1]].add(1.0)
    AT = _transpose(A)
    cs0, diag0 = _colsum_diag(A, with_diag=True)
    a2d0 = jnp.where(diag0 > 0, diag0, 2.0)
    deg0 = cs0 - diag0 + a2d0
    dinv0 = jnp.where(deg0 > 0, 1.0 / jnp.sqrt(deg0), 0.0)
    h = _conv(A, XW0, dinv0, diag0, down_b[0].reshape(1, H),
              relu=True, real=N, lvl0=True)

    xs = [h]
    As = [(A, dinv0, diag0)]
    perms = []
    A_cur, AT_cur = A, AT
    for i in range(1, depth + 1):
        prev_real, prev_pad = reals[i - 1], pads[i - 1]
        k, np_i = reals[i], pads[i]
        w = pool_w[i - 1].reshape(H, 1)
        hw = _rowdot(h, w, real=prev_real)
        score = jnp.tanh(hw[:prev_real, 0] / jnp.linalg.norm(pool_w[i - 1]))
        vals, perm = jax.lax.top_k(score, k)
        perm = perm.astype(jnp.int32)
        perm_pad = jnp.pad(perm, (0, np_i - k))
        vals_pad = jnp.pad(vals, (0, np_i - k)).reshape(np_i, 1)
        hp = _gather_scale(h, perm_pad, vals_pad, np_i)

        dt = jnp.bfloat16 if i == 1 else _F32
        R = _gather_diag1(A_cur, perm_pad, k, np_i, dt)
        Rt = _gather_diag1(AT_cur, perm_pad, k, np_i, dt)
        C = _aug_mm(R, Rt)
        cs, _ = _colsum_diag(C, with_diag=False)
        dinv = jnp.where(cs + 2.0 > 0, 1.0 / jnp.sqrt(cs + 2.0), 0.0)
        XW = _mm(hp, down_W[i])
        h = _conv(C, XW, dinv, None, down_b[i].reshape(1, H),
                  relu=True, real=k, lvl0=False)
        perms.append(perm)
        if i < depth:
            xs.append(h)
            As.append((C, dinv, None))
            AT_cur = _transpose(C)
            A_cur = C

    for ui in range(depth):
        j = depth - 1 - ui
        res = xs[j]
        A_j, dinv_j, diag_j = As[j]
        perm = perms[j]
        k_next = reals[j + 1]
        ip = jnp.full((pads[j],), -1, jnp.int32).at[perm].set(
            jnp.arange(k_next, dtype=jnp.int32))
        hsum = _unpool(res, h, ip)
        if ui < depth - 1:
            XW = _mm(hsum, up_W[ui])
            h = _conv(A_j, XW, dinv_j, None, up_b[ui].reshape(1, H),
                      relu=True, real=reals[j], lvl0=False)
        else:
            y = _rowdot(hsum, up_Wl.reshape(H, 1), real=N)
            out = _conv(A_j, y, dinv_j, diag_j, up_bl.reshape(1, 1),
                        relu=False, real=N, lvl0=True)
            return out[:N]
